# Initial kernel scaffold; baseline (speedup 1.0000x reference)
import functools

import jax
import jax.numpy as jnp
from jax import lax
from jax.experimental import pallas as pl
from jax.experimental.pallas import tpu as pltpu
from jax.experimental.pallas import tpu_sc as plsc

B = 8
H = 512
W = 512
NKP = 1000
NSORT = 1024
NC = 2
NS = 16
NWORK = NC * NS
SLICE = (B * H * W) // NWORK
CAP = 2048
KP_PER_W = (B * NSORT) // NWORK
ROWS_PER_KP = 10
IDXLEN = KP_PER_W * ROWS_PER_KP
GCHUNK = 128
PAD_IDX = jnp.int32(1 << 30)


def _nms_body(x_ref, o_ref):
    x = x_ref[0]
    ninf = jnp.full((H, 2), -jnp.inf, jnp.float32)
    xe = jnp.concatenate([ninf, x, ninf], axis=1)
    rm = x
    for i in (0, 1, 3, 4):
        rm = jnp.maximum(rm, xe[:, i:i + W])
    ninf_r = jnp.full((2, W), -jnp.inf, jnp.float32)
    re = jnp.concatenate([ninf_r, rm, ninf_r], axis=0)
    cm = rm
    for i in (0, 1, 3, 4):
        cm = jnp.maximum(cm, re[i:i + H, :])
    o_ref[0] = jnp.where(cm == x, x, 0.0)


_nms_call = pl.pallas_call(
    _nms_body,
    grid=(B,),
    in_specs=[pl.BlockSpec((1, H, W), lambda b: (b, 0, 0))],
    out_specs=pl.BlockSpec((1, H, W), lambda b: (b, 0, 0)),
    out_shape=jax.ShapeDtypeStruct((B, H, W), jnp.float32),
)


def _compact_body(filt_hbm, outv_hbm, outi_hbm, inbuf, vbuf, ibuf):
    wid = lax.axis_index("s") * NC + lax.axis_index("c")
    base = wid * SLICE
    pltpu.sync_copy(filt_hbm.at[pl.ds(base, SLICE)], inbuf)

    neg = jnp.full((16,), -jnp.inf, jnp.float32)
    padi = jnp.full((16,), PAD_IDX, jnp.int32)

    def fill(i, carry):
        vbuf[pl.ds(i * 16, 16)] = neg
        ibuf[pl.ds(i * 16, 16)] = padi
        return carry

    lax.fori_loop(0, CAP // 16, fill, jnp.int32(0))

    iota16 = lax.iota(jnp.int32, 16)

    def body(i, pos):
        v = inbuf[pl.ds(i * 16, 16)]
        m = v > 0.0
        cnt = jnp.sum(m.astype(jnp.int32))
        idx = base + i * 16 + iota16
        plsc.store_compressed(vbuf.at[pl.ds(pos, 16)], v, mask=m)
        plsc.store_compressed(ibuf.at[pl.ds(pos, 16)], idx, mask=m)
        return jnp.minimum(pos + cnt, CAP - 16)

    lax.fori_loop(0, SLICE // 16, body, jnp.int32(0))
    pltpu.sync_copy(vbuf, outv_hbm.at[wid])
    pltpu.sync_copy(ibuf, outi_hbm.at[wid])


_compact_call = functools.partial(
    pl.kernel,
    out_type=[
        jax.ShapeDtypeStruct((NWORK, CAP), jnp.float32),
        jax.ShapeDtypeStruct((NWORK, CAP), jnp.int32),
    ],
    mesh=plsc.VectorSubcoreMesh(core_axis_name="c", subcore_axis_name="s"),
    scratch_types=[
        pltpu.VMEM((SLICE,), jnp.float32),
        pltpu.VMEM((CAP,), jnp.float32),
        pltpu.VMEM((CAP,), jnp.int32),
    ],
)(_compact_body)


def _roll(a, sh):
    sh %= a.shape[1]
    if sh == 0:
        return a
    return jnp.concatenate([a[:, -sh:], a[:, :-sh]], axis=1)


def _sort_body(v_ref, i_ref, o_ref):
    v = v_ref[...]
    ix = i_ref[...]
    n = v.shape[1]
    pos = lax.broadcasted_iota(jnp.int32, v.shape, 1)
    k = 2
    while k <= n:
        j = k // 2
        while j >= 1:
            bit = (pos & j) != 0
            pv = jnp.where(bit, _roll(v, j), _roll(v, -j))
            pi = jnp.where(bit, _roll(ix, j), _roll(ix, -j))
            beats = (v > pv) | ((v == pv) & (ix < pi))
            desc = (pos & k) == 0
            take_self = beats == (desc != bit)
            v = jnp.where(take_self, v, pv)
            ix = jnp.where(take_self, ix, pi)
            j //= 2
        k *= 2
    o_ref[...] = ix[:, :NSORT]


_sort_call = pl.pallas_call(
    _sort_body,
    out_shape=jax.ShapeDtypeStruct((B, NSORT), jnp.int32),
)


def _sample_body(rows_hbm, kp_hbm, out_hbm, kpbuf, idxbuf, rowbuf, outbuf, sem):
    wid = lax.axis_index("s") * NC + lax.axis_index("c")
    b = wid // (NWORK // B)
    pltpu.sync_copy(kp_hbm.at[pl.ds(wid * KP_PER_W, KP_PER_W)], kpbuf)

    iota16 = lax.iota(jnp.int32, 16)

    def addr(t):
        kp = kpbuf[pl.ds(t * 16, 16)]
        y = (kp >> 9) & (H - 1)
        x = kp & (W - 1)
        s0 = jnp.clip(x - 2, 0, W - 5)
        r0w = jnp.minimum(s0 >> 3, W // 8 - 2)
        return kp, y, x, r0w

    def build_idx(t, carry):
        _, y, x, r0w = addr(t)
        for dy in range(5):
            ycl = jnp.clip(y + (dy - 2), 0, H - 1)
            r = (b * H + ycl) * (W // 8) + r0w
            for h in range(2):
                idxbuf[pl.ds(t * 160 + dy * 32 + h * 16, 16)] = r + h
        return carry

    lax.fori_loop(0, KP_PER_W // 16, build_idx, jnp.int32(0))

    cps = []
    for ch in range(IDXLEN // GCHUNK):
        cps.append(pltpu.async_copy(
            rows_hbm.at[idxbuf.at[pl.ds(ch * GCHUNK, GCHUNK)]],
            rowbuf.at[pl.ds(ch * GCHUNK, GCHUNK)],
            sem))
    for cp in cps:
        cp.wait()

    def sample(t, carry):
        _, y, x, r0w = addr(t)
        colbase = r0w * 8
        tb = t * 160
        xf = x.astype(jnp.float32)
        yf = y.astype(jnp.float32)

        def window(dyc, oc):
            ridx = tb + dyc * 32 + (oc >> 3) * 16 + iota16
            return plsc.load_gather(rowbuf, [ridx, oc & 7])

        xnum = jnp.zeros((16,), jnp.float32)
        ynum = jnp.zeros((16,), jnp.float32)
        den = jnp.zeros((16,), jnp.float32)
        for dy in range(-2, 3):
            yin = (y + dy >= 0) & (y + dy <= H - 1)
            for dx in range(-2, 3):
                xin = (x + dx >= 0) & (x + dx <= W - 1)
                oc = jnp.clip(x + dx, 0, W - 1) - colbase
                val = window(dy + 2, oc)
                e = jnp.where(yin & xin, jnp.exp(val * 5.0), 1.0)
                xnum = xnum + e * dx
                ynum = ynum + e * dy
                den = den + e
        kpx = xf + xnum / den
        kpy = yf + ynum / den

        def floor_ceil(u):
            ti = u.astype(jnp.int32)
            tf = ti.astype(jnp.float32)
            return ti - (u < tf).astype(jnp.int32), ti + (u > tf).astype(jnp.int32)

        fxi, cxi = floor_ceil(kpx)
        fyi, cyi = floor_ceil(kpy)
        fxc = jnp.clip(fxi, 0, W - 1)
        cxc = jnp.clip(cxi, 0, W - 1)
        fyc = jnp.clip(fyi, 0, H - 1)
        cyc = jnp.clip(cyi, 0, H - 1)

        def corner(cy, cx):
            return window(cy - y + 2, cx - colbase)

        c00 = corner(fyc, fxc)
        c10 = corner(fyc, cxc)
        c01 = corner(cyc, fxc)
        c11 = corner(cyc, cxc)
        fxf = fxc.astype(jnp.float32)
        cxf = cxc.astype(jnp.float32)
        fyf = fyc.astype(jnp.float32)
        cyf = cyc.astype(jnp.float32)
        conf = (c00 * (cxf - kpx) * (cyf - kpy)
                + c10 * (kpx - fxf) * (cyf - kpy)
                + c01 * (cxf - kpx) * (kpy - fyf)
                + c11 * (kpx - fxf) * (kpy - fyf))

        rows = t * 16 + iota16
        zero = jnp.zeros((16,), jnp.int32)
        plsc.store_scatter(outbuf, [rows, zero], kpx)
        plsc.store_scatter(outbuf, [rows, zero + 1], kpy)
        plsc.store_scatter(outbuf, [rows, zero + 2], conf)
        return carry

    lax.fori_loop(0, KP_PER_W // 16, sample, jnp.int32(0))
    pltpu.sync_copy(outbuf, out_hbm.at[pl.ds(wid * KP_PER_W, KP_PER_W)])


_sample_call = functools.partial(
    pl.kernel,
    out_type=jax.ShapeDtypeStruct((B * NSORT, 8), jnp.float32),
    mesh=plsc.VectorSubcoreMesh(core_axis_name="c", subcore_axis_name="s"),
    scratch_types=[
        pltpu.VMEM((KP_PER_W,), jnp.int32),
        pltpu.VMEM((IDXLEN,), jnp.int32),
        pltpu.VMEM((IDXLEN, 8), jnp.float32),
        pltpu.VMEM((KP_PER_W, 8), jnp.float32),
        pltpu.SemaphoreType.DMA,
    ],
)(_sample_body)


def kernel(score_map):
    s = score_map.reshape(B, H, W)
    filt = _nms_call(s)
    vals, idxs = _compact_call(filt.reshape(B * H * W))
    per_b = (NWORK // B) * CAP
    topidx = _sort_call(vals.reshape(B, per_b), idxs.reshape(B, per_b))
    rows = score_map.reshape((B * H * W) // 8, 8)
    out8 = _sample_call(rows, topidx.reshape(B * NSORT))
    return out8.reshape(B, NSORT, 8)[:, :NKP, :3]


# speedup vs baseline: 53.1832x; 53.1832x over previous
import functools

import jax
import jax.numpy as jnp
import numpy as np
from jax import lax
from jax.experimental import pallas as pl
from jax.experimental.pallas import tpu as pltpu
from jax.experimental.pallas import tpu_sc as plsc

B = 8
H = 512
W = 512
NKP = 1000
NSORT = 1024
NC = 2
NS = 16
NWORK = NC * NS
SLICE = (B * H * W) // NWORK
CAP = 4096
KP_PER_W = (B * NSORT) // NWORK
ROWS_PER_KP = 10
IDXLEN = KP_PER_W * ROWS_PER_KP
GCHUNK = 128
PAD_IDX = 1 << 30


def _nms_body(x_ref, o_ref):
    x = x_ref[0]
    ninf = jnp.full((H, 2), -jnp.inf, jnp.float32)
    xe = jnp.concatenate([ninf, x, ninf], axis=1)
    rm = x
    for i in (0, 1, 3, 4):
        rm = jnp.maximum(rm, xe[:, i:i + W])
    ninf_r = jnp.full((2, W), -jnp.inf, jnp.float32)
    re = jnp.concatenate([ninf_r, rm, ninf_r], axis=0)
    cm = rm
    for i in (0, 1, 3, 4):
        cm = jnp.maximum(cm, re[i:i + H, :])
    o_ref[0] = jnp.where(cm == x, x, 0.0)


_nms_call = pl.pallas_call(
    _nms_body,
    grid=(B,),
    in_specs=[pl.BlockSpec((1, H, W), lambda b: (b, 0, 0))],
    out_specs=pl.BlockSpec((1, H, W), lambda b: (b, 0, 0)),
    out_shape=jax.ShapeDtypeStruct((B, H, W), jnp.float32),
)


def _compact_body(filt_hbm, outv_hbm, outi_hbm, inbuf, vbuf, ibuf):
    wid = lax.axis_index("s") * NC + lax.axis_index("c")
    base = wid * SLICE
    pltpu.sync_copy(filt_hbm.at[pl.ds(base, SLICE)], inbuf)

    neg = jnp.full((16,), -jnp.inf, jnp.float32)
    padi = jnp.full((16,), PAD_IDX, jnp.int32)

    def fill(i, carry):
        vbuf[pl.ds(i * 16, 16)] = neg
        ibuf[pl.ds(i * 16, 16)] = padi
        return carry

    lax.fori_loop(0, (CAP + 16) // 16, fill, jnp.int32(0))

    iota16 = lax.iota(jnp.int32, 16)

    def body(i, pos):
        v = inbuf[pl.ds(i * 16, 16)]
        m = v > 0.0
        rank = plsc.cumsum(m.astype(jnp.int32))
        cnt = plsc.all_reduce_population_count(m)
        tgt = jnp.where(m, pos + rank - 1, CAP)
        idx = base + i * 16 + iota16
        plsc.store_scatter(vbuf, [tgt], v)
        plsc.store_scatter(ibuf, [tgt], idx)
        return jnp.minimum(pos + cnt, CAP - 16)

    lax.fori_loop(0, SLICE // 16, body, jnp.full((16,), 0, jnp.int32))
    pltpu.sync_copy(vbuf.at[pl.ds(0, CAP)], outv_hbm.at[wid])
    pltpu.sync_copy(ibuf.at[pl.ds(0, CAP)], outi_hbm.at[wid])


@functools.cache
def _compact_call():
    return functools.partial(
        pl.kernel,
        out_type=[
            jax.ShapeDtypeStruct((NWORK, CAP), jnp.float32),
            jax.ShapeDtypeStruct((NWORK, CAP), jnp.int32),
        ],
        mesh=plsc.VectorSubcoreMesh(core_axis_name="c", subcore_axis_name="s"),
        compiler_params=pltpu.CompilerParams(needs_layout_passes=False),
        scratch_types=[
            pltpu.VMEM((SLICE,), jnp.float32),
            pltpu.VMEM((CAP + 16,), jnp.float32),
            pltpu.VMEM((CAP + 16,), jnp.int32),
        ],
    )(_compact_body)


NTOT = (NWORK // B) * CAP


def _stage_table():
    st = []
    k = 2
    while k <= NSORT:
        ka = k if k < NSORT else 2 * NTOT
        j = k // 2
        while j >= 1:
            st.append((j, ka, NTOT // 2))
            j //= 2
        k *= 2
    n = NTOT
    while n > NSORT:
        st.append((n // 2, 2 * NTOT, 2 * NTOT))
        n //= 2
        ka = (n // 2) if n > NSORT else 2 * NTOT
        j = NSORT // 2
        while j >= 1:
            st.append((j, ka, 2 * NTOT))
            j //= 2
    return st


_STAGES = _stage_table()
NSTAGE = len(_STAGES)
_STAGE_PARAMS = np.array(_STAGES, dtype=np.int32).T.copy()


def _sort_body(p_ref, v_ref, i_ref, o_ref, vs, ixs):
    s = pl.program_id(0)

    @pl.when(s == 0)
    def _():
        vs[...] = v_ref[...]
        ixs[...] = i_ref[...]

    j = p_ref[0, s]
    ka = p_ref[1, s]
    kb = p_ref[2, s]
    v = vs[...]
    ix = ixs[...]
    pos = lax.broadcasted_iota(jnp.int32, v.shape, 1)
    bit = (pos & j) != 0
    rvj = pltpu.roll(v, j, 1)
    rij = pltpu.roll(ix, j, 1)
    pv = jnp.where(bit, rvj, pltpu.roll(rvj, NTOT - 2 * j, 1))
    pi = jnp.where(bit, rij, pltpu.roll(rij, NTOT - 2 * j, 1))
    beats = (v > pv) | ((v == pv) & (ix < pi))
    desc = ((pos & ka) == 0) == ((pos & kb) == 0)
    take_self = beats == (desc != bit)
    vs[...] = jnp.where(take_self, v, pv)
    ixs[...] = jnp.where(take_self, ix, pi)

    @pl.when(s == NSTAGE - 1)
    def _():
        o_ref[...] = ixs[:, :NSORT]


_sort_call = pl.pallas_call(
    _sort_body,
    grid=(NSTAGE,),
    in_specs=[
        pl.BlockSpec(memory_space=pltpu.SMEM),
        pl.BlockSpec((B, NTOT), lambda s: (0, 0)),
        pl.BlockSpec((B, NTOT), lambda s: (0, 0)),
    ],
    out_specs=pl.BlockSpec((B, NSORT), lambda s: (0, 0)),
    out_shape=jax.ShapeDtypeStruct((B, NSORT), jnp.int32),
    scratch_shapes=[
        pltpu.VMEM((B, NTOT), jnp.float32),
        pltpu.VMEM((B, NTOT), jnp.int32),
    ],
)


def _sample_body(rows_hbm, kp_hbm, out_hbm, kpbuf, idxbuf, rowbuf, outbuf, sem):
    wid = lax.axis_index("s") * NC + lax.axis_index("c")
    b = wid // (NWORK // B)
    pltpu.sync_copy(kp_hbm.at[pl.ds(wid * KP_PER_W, KP_PER_W)], kpbuf)

    iota16 = lax.iota(jnp.int32, 16)

    def addr(t):
        kp = kpbuf[pl.ds(t * 16, 16)]
        y = (kp >> 9) & (H - 1)
        x = kp & (W - 1)
        s0 = jnp.clip(x - 2, 0, W - 5)
        r0w = jnp.minimum(s0 >> 3, W // 8 - 2)
        return kp, y, x, r0w

    def build_idx(t, carry):
        _, y, x, r0w = addr(t)
        for dy in range(5):
            ycl = jnp.clip(y + (dy - 2), 0, H - 1)
            r = (b * H + ycl) * (W // 8) + r0w
            for h in range(2):
                idxbuf[pl.ds(t * 160 + dy * 32 + h * 16, 16)] = r + h
        return carry

    lax.fori_loop(0, KP_PER_W // 16, build_idx, jnp.int32(0))

    cps = []
    for ch in range(IDXLEN // GCHUNK):
        cps.append(pltpu.async_copy(
            rows_hbm.at[idxbuf.at[pl.ds(ch * GCHUNK, GCHUNK)]],
            rowbuf.at[pl.ds(ch * GCHUNK, GCHUNK)],
            sem))
    for cp in cps:
        cp.wait()

    def sample(t, carry):
        _, y, x, r0w = addr(t)
        colbase = r0w * 8
        tb = t * 160
        xf = x.astype(jnp.float32)
        yf = y.astype(jnp.float32)

        def window(dyc, oc):
            ridx = tb + dyc * 32 + (oc >> 3) * 16 + iota16
            return plsc.load_gather(rowbuf, [ridx, oc & 7])

        xnum = jnp.zeros((16,), jnp.float32)
        ynum = jnp.zeros((16,), jnp.float32)
        den = jnp.zeros((16,), jnp.float32)
        for dy in range(-2, 3):
            yin = (y + dy >= 0) & (y + dy <= H - 1)
            for dx in range(-2, 3):
                xin = (x + dx >= 0) & (x + dx <= W - 1)
                oc = jnp.clip(x + dx, 0, W - 1) - colbase
                val = window(dy + 2, oc)
                e = jnp.where(yin & xin, jnp.exp(val * 5.0), 1.0)
                xnum = xnum + e * dx
                ynum = ynum + e * dy
                den = den + e
        kpx = xf + xnum / den
        kpy = yf + ynum / den

        def floor_ceil(u):
            ti = u.astype(jnp.int32)
            tf = ti.astype(jnp.float32)
            return ti - (u < tf).astype(jnp.int32), ti + (u > tf).astype(jnp.int32)

        fxi, cxi = floor_ceil(kpx)
        fyi, cyi = floor_ceil(kpy)
        fxc = jnp.clip(fxi, 0, W - 1)
        cxc = jnp.clip(cxi, 0, W - 1)
        fyc = jnp.clip(fyi, 0, H - 1)
        cyc = jnp.clip(cyi, 0, H - 1)

        def corner(cy, cx):
            return window(cy - y + 2, cx - colbase)

        c00 = corner(fyc, fxc)
        c10 = corner(fyc, cxc)
        c01 = corner(cyc, fxc)
        c11 = corner(cyc, cxc)
        fxf = fxc.astype(jnp.float32)
        cxf = cxc.astype(jnp.float32)
        fyf = fyc.astype(jnp.float32)
        cyf = cyc.astype(jnp.float32)
        conf = (c00 * (cxf - kpx) * (cyf - kpy)
                + c10 * (kpx - fxf) * (cyf - kpy)
                + c01 * (cxf - kpx) * (kpy - fyf)
                + c11 * (kpx - fxf) * (kpy - fyf))

        rows = t * 16 + iota16
        zero = jnp.zeros((16,), jnp.int32)
        plsc.store_scatter(outbuf, [rows, zero], kpx)
        plsc.store_scatter(outbuf, [rows, zero + 1], kpy)
        plsc.store_scatter(outbuf, [rows, zero + 2], conf)
        return carry

    lax.fori_loop(0, KP_PER_W // 16, sample, jnp.int32(0))
    pltpu.sync_copy(outbuf, out_hbm.at[pl.ds(wid * KP_PER_W, KP_PER_W)])


@functools.cache
def _sample_call():
    return functools.partial(
        pl.kernel,
        out_type=jax.ShapeDtypeStruct((B * NSORT, 8), jnp.float32),
        mesh=plsc.VectorSubcoreMesh(core_axis_name="c", subcore_axis_name="s"),
        compiler_params=pltpu.CompilerParams(
            needs_layout_passes=False, use_tc_tiling_on_sc=False),
        scratch_types=[
            pltpu.VMEM((KP_PER_W,), jnp.int32),
            pltpu.VMEM((IDXLEN,), jnp.int32),
            pltpu.VMEM((IDXLEN, 8), jnp.float32),
            pltpu.VMEM((KP_PER_W, 8), jnp.float32),
            pltpu.SemaphoreType.DMA,
        ],
    )(_sample_body)


def kernel(score_map):
    s = score_map.reshape(B, H, W)
    filt = _nms_call(s)
    per_b = (NWORK // B) * CAP
    vals, idxs = _compact_call()(filt.reshape(B * H * W))
    topidx = _sort_call(jnp.asarray(_STAGE_PARAMS),
                        vals.reshape(B, per_b), idxs.reshape(B, per_b))
    rows = score_map.reshape((B * H * W) // 8, 8)
    out8 = _sample_call()(rows, topidx.reshape(B * NSORT))
    return out8.reshape(B, NSORT, 8)[:, :NKP, :3]


# speedup vs baseline: 65.2167x; 1.2263x over previous
import functools

import jax
import jax.numpy as jnp
import numpy as np
from jax import lax
from jax.experimental import pallas as pl
from jax.experimental.pallas import tpu as pltpu
from jax.experimental.pallas import tpu_sc as plsc

B = 8
H = 512
W = 512
NKP = 1000
NSORT = 1024
NC = 2
NS = 16
NWORK = NC * NS
SLICE = (B * H * W) // NWORK
CAP = 4096
KP_PER_W = (B * NSORT) // NWORK
ROWS_PER_KP = 10
IDXLEN = KP_PER_W * ROWS_PER_KP
GCHUNK = 128
PAD_IDX = 1 << 30


def _nms_body(x_ref, o_ref):
    x = x_ref[0]
    ninf = jnp.full((H, 2), -jnp.inf, jnp.float32)
    xe = jnp.concatenate([ninf, x, ninf], axis=1)
    rm = x
    for i in (0, 1, 3, 4):
        rm = jnp.maximum(rm, xe[:, i:i + W])
    ninf_r = jnp.full((2, W), -jnp.inf, jnp.float32)
    re = jnp.concatenate([ninf_r, rm, ninf_r], axis=0)
    cm = rm
    for i in (0, 1, 3, 4):
        cm = jnp.maximum(cm, re[i:i + H, :])
    o_ref[0] = jnp.where(cm == x, x, 0.0)


_nms_call = pl.pallas_call(
    _nms_body,
    grid=(B,),
    in_specs=[pl.BlockSpec((1, H, W), lambda b: (b, 0, 0))],
    out_specs=pl.BlockSpec((1, H, W), lambda b: (b, 0, 0)),
    out_shape=jax.ShapeDtypeStruct((B, H, W), jnp.float32),
)


def _compact_body(filt_hbm, outv_hbm, outi_hbm, inbuf, vbuf, ibuf):
    wid = lax.axis_index("s") * NC + lax.axis_index("c")
    base = wid * SLICE
    pltpu.sync_copy(filt_hbm.at[pl.ds(base, SLICE)], inbuf)

    neg = jnp.full((16,), -jnp.inf, jnp.float32)
    padi = jnp.full((16,), PAD_IDX, jnp.int32)

    def fill(i, carry):
        vbuf[pl.ds(i * 16, 16)] = neg
        ibuf[pl.ds(i * 16, 16)] = padi
        return carry

    lax.fori_loop(0, (CAP + 16) // 16, fill, jnp.int32(0))

    iota16 = lax.iota(jnp.int32, 16)

    def body(i, pos):
        vs_ = [inbuf[pl.ds((i * 4 + g) * 16, 16)] for g in range(4)]
        ms_ = [v > 0.0 for v in vs_]
        ranks = [plsc.cumsum(m.astype(jnp.int32)) for m in ms_]
        cnts = [plsc.all_reduce_population_count(m) for m in ms_]
        for g in range(4):
            tgt = jnp.where(ms_[g], pos + ranks[g] - 1, CAP)
            idx = base + (i * 4 + g) * 16 + iota16
            plsc.store_scatter(vbuf, [tgt], vs_[g])
            plsc.store_scatter(ibuf, [tgt], idx)
            pos = jnp.minimum(pos + cnts[g], CAP - 16)
        return pos

    lax.fori_loop(0, SLICE // 64, body, jnp.full((16,), 0, jnp.int32))
    pltpu.sync_copy(vbuf.at[pl.ds(0, CAP)], outv_hbm.at[wid])
    pltpu.sync_copy(ibuf.at[pl.ds(0, CAP)], outi_hbm.at[wid])


@functools.cache
def _compact_call():
    return functools.partial(
        pl.kernel,
        out_type=[
            jax.ShapeDtypeStruct((NWORK, CAP), jnp.float32),
            jax.ShapeDtypeStruct((NWORK, CAP), jnp.int32),
        ],
        mesh=plsc.VectorSubcoreMesh(core_axis_name="c", subcore_axis_name="s"),
        compiler_params=pltpu.CompilerParams(needs_layout_passes=False),
        scratch_types=[
            pltpu.VMEM((SLICE,), jnp.float32),
            pltpu.VMEM((CAP + 16,), jnp.float32),
            pltpu.VMEM((CAP + 16,), jnp.int32),
        ],
    )(_compact_body)


NTOT = (NWORK // B) * CAP


def _phase_table():
    ph = []
    k = 2
    while k <= NSORT:
        ka = k if k < NSORT else 2 * NTOT
        nj = k.bit_length() - 1
        ph.append((k // 2, nj, ka, NTOT // 2))
        k *= 2
    n = NTOT
    while n > NSORT:
        ph.append((n // 2, 1, 2 * NTOT, 2 * NTOT))
        n //= 2
        ka = (n // 2) if n > NSORT else 2 * NTOT
        ph.append((NSORT // 2, 10, ka, 2 * NTOT))
    return ph


_PHASES = _phase_table()
NPHASE = len(_PHASES)
_STAGE_PARAMS = np.array(_PHASES, dtype=np.int32).T.copy()


def _sort_body(p_ref, v_ref, i_ref, o_ref, vs, ixs):
    s = pl.program_id(0)

    @pl.when(s == 0)
    def _():
        vs[...] = v_ref[...]
        ixs[...] = i_ref[...]

    j0 = p_ref[0, s]
    nj = p_ref[1, s]
    ka = p_ref[2, s]
    kb = p_ref[3, s]
    pos = lax.broadcasted_iota(jnp.int32, (B, NTOT), 1)
    desc = ((pos & ka) == 0) == ((pos & kb) == 0)

    def stage(i, carry):
        v, ix = carry
        j = lax.shift_right_logical(j0, i)
        bit = (pos & j) != 0
        rvj = pltpu.roll(v, j, 1)
        rij = pltpu.roll(ix, j, 1)
        pv = jnp.where(bit, rvj, pltpu.roll(rvj, NTOT - 2 * j, 1))
        pi = jnp.where(bit, rij, pltpu.roll(rij, NTOT - 2 * j, 1))
        beats = (v > pv) | ((v == pv) & (ix < pi))
        take_self = beats == (desc != bit)
        return jnp.where(take_self, v, pv), jnp.where(take_self, ix, pi)

    v, ix = lax.fori_loop(0, nj, stage, (vs[...], ixs[...]))
    vs[...] = v
    ixs[...] = ix

    @pl.when(s == NPHASE - 1)
    def _():
        o_ref[...] = ix[:, :NSORT]


_sort_call = pl.pallas_call(
    _sort_body,
    grid=(NPHASE,),
    in_specs=[
        pl.BlockSpec(memory_space=pltpu.SMEM),
        pl.BlockSpec((B, NTOT), lambda s: (0, 0)),
        pl.BlockSpec((B, NTOT), lambda s: (0, 0)),
    ],
    out_specs=pl.BlockSpec((B, NSORT), lambda s: (0, 0)),
    out_shape=jax.ShapeDtypeStruct((B, NSORT), jnp.int32),
    scratch_shapes=[
        pltpu.VMEM((B, NTOT), jnp.float32),
        pltpu.VMEM((B, NTOT), jnp.int32),
    ],
)


def _sample_body(rows_hbm, kp_hbm, out_hbm, kpbuf, idxbuf, rowbuf, outbuf, sem):
    wid = lax.axis_index("s") * NC + lax.axis_index("c")
    b = wid // (NWORK // B)
    pltpu.sync_copy(kp_hbm.at[pl.ds(wid * KP_PER_W, KP_PER_W)], kpbuf)

    iota16 = lax.iota(jnp.int32, 16)

    def addr(t):
        kp = kpbuf[pl.ds(t * 16, 16)]
        y = (kp >> 9) & (H - 1)
        x = kp & (W - 1)
        s0 = jnp.clip(x - 2, 0, W - 5)
        r0w = jnp.minimum(s0 >> 3, W // 8 - 2)
        return kp, y, x, r0w

    def build_idx(t, carry):
        _, y, x, r0w = addr(t)
        for dy in range(5):
            ycl = jnp.clip(y + (dy - 2), 0, H - 1)
            r = (b * H + ycl) * (W // 8) + r0w
            for h in range(2):
                idxbuf[pl.ds(t * 160 + dy * 32 + h * 16, 16)] = r + h
        return carry

    lax.fori_loop(0, KP_PER_W // 16, build_idx, jnp.int32(0))

    cps = []
    for ch in range(IDXLEN // GCHUNK):
        cps.append(pltpu.async_copy(
            rows_hbm.at[idxbuf.at[pl.ds(ch * GCHUNK, GCHUNK)]],
            rowbuf.at[pl.ds(ch * GCHUNK, GCHUNK)],
            sem))
    for cp in cps:
        cp.wait()

    def sample(t, carry):
        _, y, x, r0w = addr(t)
        colbase = r0w * 8
        tb = t * 160
        xf = x.astype(jnp.float32)
        yf = y.astype(jnp.float32)

        def window(dyc, oc):
            ridx = tb + dyc * 32 + (oc >> 3) * 16 + iota16
            return plsc.load_gather(rowbuf, [ridx, oc & 7])

        xnum = jnp.zeros((16,), jnp.float32)
        ynum = jnp.zeros((16,), jnp.float32)
        den = jnp.zeros((16,), jnp.float32)
        for dy in range(-2, 3):
            yin = (y + dy >= 0) & (y + dy <= H - 1)
            for dx in range(-2, 3):
                xin = (x + dx >= 0) & (x + dx <= W - 1)
                oc = jnp.clip(x + dx, 0, W - 1) - colbase
                val = window(dy + 2, oc)
                e = jnp.where(yin & xin, jnp.exp(val * 5.0), 1.0)
                xnum = xnum + e * dx
                ynum = ynum + e * dy
                den = den + e
        kpx = xf + xnum / den
        kpy = yf + ynum / den

        def floor_ceil(u):
            ti = u.astype(jnp.int32)
            tf = ti.astype(jnp.float32)
            return ti - (u < tf).astype(jnp.int32), ti + (u > tf).astype(jnp.int32)

        fxi, cxi = floor_ceil(kpx)
        fyi, cyi = floor_ceil(kpy)
        fxc = jnp.clip(fxi, 0, W - 1)
        cxc = jnp.clip(cxi, 0, W - 1)
        fyc = jnp.clip(fyi, 0, H - 1)
        cyc = jnp.clip(cyi, 0, H - 1)

        def corner(cy, cx):
            return window(cy - y + 2, cx - colbase)

        c00 = corner(fyc, fxc)
        c10 = corner(fyc, cxc)
        c01 = corner(cyc, fxc)
        c11 = corner(cyc, cxc)
        fxf = fxc.astype(jnp.float32)
        cxf = cxc.astype(jnp.float32)
        fyf = fyc.astype(jnp.float32)
        cyf = cyc.astype(jnp.float32)
        conf = (c00 * (cxf - kpx) * (cyf - kpy)
                + c10 * (kpx - fxf) * (cyf - kpy)
                + c01 * (cxf - kpx) * (kpy - fyf)
                + c11 * (kpx - fxf) * (kpy - fyf))

        rows = t * 16 + iota16
        zero = jnp.zeros((16,), jnp.int32)
        plsc.store_scatter(outbuf, [rows, zero], kpx)
        plsc.store_scatter(outbuf, [rows, zero + 1], kpy)
        plsc.store_scatter(outbuf, [rows, zero + 2], conf)
        return carry

    lax.fori_loop(0, KP_PER_W // 16, sample, jnp.int32(0))
    pltpu.sync_copy(outbuf, out_hbm.at[pl.ds(wid * KP_PER_W, KP_PER_W)])


@functools.cache
def _sample_call():
    return functools.partial(
        pl.kernel,
        out_type=jax.ShapeDtypeStruct((B * NSORT, 8), jnp.float32),
        mesh=plsc.VectorSubcoreMesh(core_axis_name="c", subcore_axis_name="s"),
        compiler_params=pltpu.CompilerParams(
            needs_layout_passes=False, use_tc_tiling_on_sc=False),
        scratch_types=[
            pltpu.VMEM((KP_PER_W,), jnp.int32),
            pltpu.VMEM((IDXLEN,), jnp.int32),
            pltpu.VMEM((IDXLEN, 8), jnp.float32),
            pltpu.VMEM((KP_PER_W, 8), jnp.float32),
            pltpu.SemaphoreType.DMA,
        ],
    )(_sample_body)


def kernel(score_map):
    s = score_map.reshape(B, H, W)
    filt = _nms_call(s)
    per_b = (NWORK // B) * CAP
    vals, idxs = _compact_call()(filt.reshape(B * H * W))
    topidx = _sort_call(jnp.asarray(_STAGE_PARAMS),
                        vals.reshape(B, per_b), idxs.reshape(B, per_b))
    rows = score_map.reshape((B * H * W) // 8, 8)
    out8 = _sample_call()(rows, topidx.reshape(B * NSORT))
    return out8.reshape(B, NSORT, 8)[:, :NKP, :3]


# speedup vs baseline: 107.8512x; 1.6537x over previous
import functools

import jax
import jax.numpy as jnp
import numpy as np
from jax import lax
from jax.experimental import pallas as pl
from jax.experimental.pallas import tpu as pltpu
from jax.experimental.pallas import tpu_sc as plsc

B = 8
H = 512
W = 512
NKP = 1000
NSORT = 1024
NC = 2
NS = 16
NWORK = NC * NS
SLICE = (B * H * W) // NWORK
CAP = 4096
KP_PER_W = (B * NSORT) // NWORK
ROWS_PER_KP = 10
IDXLEN = KP_PER_W * ROWS_PER_KP
GCHUNK = 128
PAD_IDX = 1 << 30


def _nms_body(x_ref, o_ref):
    x = x_ref[0]
    ninf = jnp.full((H, 2), -jnp.inf, jnp.float32)
    xe = jnp.concatenate([ninf, x, ninf], axis=1)
    rm = x
    for i in (0, 1, 3, 4):
        rm = jnp.maximum(rm, xe[:, i:i + W])
    ninf_r = jnp.full((2, W), -jnp.inf, jnp.float32)
    re = jnp.concatenate([ninf_r, rm, ninf_r], axis=0)
    cm = rm
    for i in (0, 1, 3, 4):
        cm = jnp.maximum(cm, re[i:i + H, :])
    o_ref[0] = jnp.where(cm == x, x, 0.0)


_nms_call = pl.pallas_call(
    _nms_body,
    grid=(B,),
    in_specs=[pl.BlockSpec((1, H, W), lambda b: (b, 0, 0))],
    out_specs=pl.BlockSpec((1, H, W), lambda b: (b, 0, 0)),
    out_shape=jax.ShapeDtypeStruct((B, H, W), jnp.float32),
)


def _compact_body(filt_hbm, outv_hbm, outi_hbm, inbuf, vbuf, ibuf):
    wid = lax.axis_index("s") * NC + lax.axis_index("c")
    base = wid * SLICE
    pltpu.sync_copy(filt_hbm.at[pl.ds(base, SLICE)], inbuf)

    neg = jnp.full((16,), -jnp.inf, jnp.float32)
    padi = jnp.full((16,), PAD_IDX, jnp.int32)

    def fill(i, carry):
        vbuf[pl.ds(i * 16, 16)] = neg
        ibuf[pl.ds(i * 16, 16)] = padi
        return carry

    lax.fori_loop(0, (CAP + 16) // 16, fill, jnp.int32(0))

    iota16 = lax.iota(jnp.int32, 16)

    def body(i, pos):
        vs_ = [inbuf[pl.ds((i * 4 + g) * 16, 16)] for g in range(4)]
        ms_ = [v > 0.0 for v in vs_]
        ranks = [plsc.cumsum(m.astype(jnp.int32)) for m in ms_]
        cnts = [plsc.all_reduce_population_count(m) for m in ms_]
        for g in range(4):
            tgt = jnp.where(ms_[g], pos + ranks[g] - 1, CAP)
            idx = base + (i * 4 + g) * 16 + iota16
            plsc.store_scatter(vbuf, [tgt], vs_[g])
            plsc.store_scatter(ibuf, [tgt], idx)
            pos = jnp.minimum(pos + cnts[g], CAP - 16)
        return pos

    lax.fori_loop(0, SLICE // 64, body, jnp.full((16,), 0, jnp.int32))
    pltpu.sync_copy(vbuf.at[pl.ds(0, CAP)], outv_hbm.at[wid])
    pltpu.sync_copy(ibuf.at[pl.ds(0, CAP)], outi_hbm.at[wid])


@functools.cache
def _compact_call():
    return functools.partial(
        pl.kernel,
        out_type=[
            jax.ShapeDtypeStruct((NWORK, CAP), jnp.float32),
            jax.ShapeDtypeStruct((NWORK, CAP), jnp.int32),
        ],
        mesh=plsc.VectorSubcoreMesh(core_axis_name="c", subcore_axis_name="s"),
        compiler_params=pltpu.CompilerParams(needs_layout_passes=False),
        scratch_types=[
            pltpu.VMEM((SLICE,), jnp.float32),
            pltpu.VMEM((CAP + 16,), jnp.float32),
            pltpu.VMEM((CAP + 16,), jnp.int32),
        ],
    )(_compact_body)


NTOT = (NWORK // B) * CAP


def _roll(a, sh):
    sh %= a.shape[1]
    if sh == 0:
        return a
    return jnp.concatenate([a[:, -sh:], a[:, :-sh]], axis=1)


def _cmp_exchange(v, ix, j, desc):
    pos = lax.broadcasted_iota(jnp.int32, v.shape, 1)
    bit = (pos & j) != 0
    rvj = _roll(v, j)
    rij = _roll(ix, j)
    n = v.shape[1]
    pv = jnp.where(bit, rvj, _roll(rvj, n - 2 * j))
    pi = jnp.where(bit, rij, _roll(rij, n - 2 * j))
    beats = (v > pv) | ((v == pv) & (ix < pi))
    take_self = beats == (desc != bit)
    return jnp.where(take_self, v, pv), jnp.where(take_self, ix, pi)


def _make_phase1_call(ks):
    def body(v_ref, i_ref, ov_ref, oi_ref):
        v = v_ref[...]
        ix = i_ref[...]
        pos = lax.broadcasted_iota(jnp.int32, v.shape, 1)
        lpos = pos & (NSORT - 1)
        chunk_desc = (pos & (NTOT // 2)) == 0
        for k in ks:
            desc = ((lpos & k) == 0) == chunk_desc
            j = k // 2
            while j >= 1:
                v, ix = _cmp_exchange(v, ix, j, desc)
                j //= 2
        ov_ref[...] = v
        oi_ref[...] = ix

    return pl.pallas_call(
        body,
        out_shape=[jax.ShapeDtypeStruct((B, NTOT), jnp.float32),
                   jax.ShapeDtypeStruct((B, NTOT), jnp.int32)])


def _make_fold_merge_call(n):
    h = n // 2

    def body(v_ref, i_ref, ov_ref, oi_ref):
        a, ai = v_ref[:, :h], i_ref[:, :h]
        bv, bi = v_ref[:, h:], i_ref[:, h:]
        beats = (a > bv) | ((a == bv) & (ai < bi))
        v = jnp.where(beats, a, bv)
        ix = jnp.where(beats, ai, bi)
        pos = lax.broadcasted_iota(jnp.int32, (B, h), 1)
        desc = ((pos & (h // 2)) == 0) if h > NSORT else jnp.bool_(True)
        j = NSORT // 2
        while j >= 1:
            v, ix = _cmp_exchange(v, ix, j, desc)
            j //= 2
        ov_ref[...] = v
        oi_ref[...] = ix

    return pl.pallas_call(
        body,
        out_shape=[jax.ShapeDtypeStruct((B, h), jnp.float32),
                   jax.ShapeDtypeStruct((B, h), jnp.int32)])


_P1_GROUPS = [(2, 4, 8), (16, 32), (64,), (128,), (256,), (512,), (1024,)]


@functools.cache
def _sort_calls():
    p1 = [_make_phase1_call(ks) for ks in _P1_GROUPS]
    p2 = [_make_fold_merge_call(n) for n in (NTOT, NTOT // 2, NTOT // 4,
                                             NTOT // 8)]
    return p1, p2


def _sort_topk(vals, idxs):
    p1, p2 = _sort_calls()
    v, ix = vals, idxs
    for call in p1:
        v, ix = call(v, ix)
    for call in p2:
        v, ix = call(v, ix)
    return ix


def _sample_body(rows_hbm, kp_hbm, out_hbm, kpbuf, idxbuf, rowbuf, outbuf, sem):
    wid = lax.axis_index("s") * NC + lax.axis_index("c")
    b = wid // (NWORK // B)
    pltpu.sync_copy(kp_hbm.at[pl.ds(wid * KP_PER_W, KP_PER_W)], kpbuf)

    iota16 = lax.iota(jnp.int32, 16)

    def addr(t):
        kp = kpbuf[pl.ds(t * 16, 16)]
        y = (kp >> 9) & (H - 1)
        x = kp & (W - 1)
        s0 = jnp.clip(x - 2, 0, W - 5)
        r0w = jnp.minimum(s0 >> 3, W // 8 - 2)
        return kp, y, x, r0w

    def build_idx(t, carry):
        _, y, x, r0w = addr(t)
        for dy in range(5):
            ycl = jnp.clip(y + (dy - 2), 0, H - 1)
            r = (b * H + ycl) * (W // 8) + r0w
            for h in range(2):
                idxbuf[pl.ds(t * 160 + dy * 32 + h * 16, 16)] = r + h
        return carry

    lax.fori_loop(0, KP_PER_W // 16, build_idx, jnp.int32(0))

    cps = []
    for ch in range(IDXLEN // GCHUNK):
        cps.append(pltpu.async_copy(
            rows_hbm.at[idxbuf.at[pl.ds(ch * GCHUNK, GCHUNK)]],
            rowbuf.at[pl.ds(ch * GCHUNK, GCHUNK)],
            sem))
    for cp in cps:
        cp.wait()

    def sample(t, carry):
        _, y, x, r0w = addr(t)
        colbase = r0w * 8
        tb = t * 160
        xf = x.astype(jnp.float32)
        yf = y.astype(jnp.float32)

        def window(dyc, oc):
            ridx = tb + dyc * 32 + (oc >> 3) * 16 + iota16
            return plsc.load_gather(rowbuf, [ridx, oc & 7])

        xnum = jnp.zeros((16,), jnp.float32)
        ynum = jnp.zeros((16,), jnp.float32)
        den = jnp.zeros((16,), jnp.float32)
        for dy in range(-2, 3):
            yin = (y + dy >= 0) & (y + dy <= H - 1)
            for dx in range(-2, 3):
                xin = (x + dx >= 0) & (x + dx <= W - 1)
                oc = jnp.clip(x + dx, 0, W - 1) - colbase
                val = window(dy + 2, oc)
                e = jnp.where(yin & xin, jnp.exp(val * 5.0), 1.0)
                xnum = xnum + e * dx
                ynum = ynum + e * dy
                den = den + e
        kpx = xf + xnum / den
        kpy = yf + ynum / den

        def floor_ceil(u):
            ti = u.astype(jnp.int32)
            tf = ti.astype(jnp.float32)
            return ti - (u < tf).astype(jnp.int32), ti + (u > tf).astype(jnp.int32)

        fxi, cxi = floor_ceil(kpx)
        fyi, cyi = floor_ceil(kpy)
        fxc = jnp.clip(fxi, 0, W - 1)
        cxc = jnp.clip(cxi, 0, W - 1)
        fyc = jnp.clip(fyi, 0, H - 1)
        cyc = jnp.clip(cyi, 0, H - 1)

        def corner(cy, cx):
            return window(cy - y + 2, cx - colbase)

        c00 = corner(fyc, fxc)
        c10 = corner(fyc, cxc)
        c01 = corner(cyc, fxc)
        c11 = corner(cyc, cxc)
        fxf = fxc.astype(jnp.float32)
        cxf = cxc.astype(jnp.float32)
        fyf = fyc.astype(jnp.float32)
        cyf = cyc.astype(jnp.float32)
        conf = (c00 * (cxf - kpx) * (cyf - kpy)
                + c10 * (kpx - fxf) * (cyf - kpy)
                + c01 * (cxf - kpx) * (kpy - fyf)
                + c11 * (kpx - fxf) * (kpy - fyf))

        rows = t * 16 + iota16
        zero = jnp.zeros((16,), jnp.int32)
        plsc.store_scatter(outbuf, [rows, zero], kpx)
        plsc.store_scatter(outbuf, [rows, zero + 1], kpy)
        plsc.store_scatter(outbuf, [rows, zero + 2], conf)
        return carry

    lax.fori_loop(0, KP_PER_W // 16, sample, jnp.int32(0))
    pltpu.sync_copy(outbuf, out_hbm.at[pl.ds(wid * KP_PER_W, KP_PER_W)])


@functools.cache
def _sample_call():
    return functools.partial(
        pl.kernel,
        out_type=jax.ShapeDtypeStruct((B * NSORT, 8), jnp.float32),
        mesh=plsc.VectorSubcoreMesh(core_axis_name="c", subcore_axis_name="s"),
        compiler_params=pltpu.CompilerParams(
            needs_layout_passes=False, use_tc_tiling_on_sc=False),
        scratch_types=[
            pltpu.VMEM((KP_PER_W,), jnp.int32),
            pltpu.VMEM((IDXLEN,), jnp.int32),
            pltpu.VMEM((IDXLEN, 8), jnp.float32),
            pltpu.VMEM((KP_PER_W, 8), jnp.float32),
            pltpu.SemaphoreType.DMA,
        ],
    )(_sample_body)


def kernel(score_map):
    s = score_map.reshape(B, H, W)
    filt = _nms_call(s)
    per_b = (NWORK // B) * CAP
    vals, idxs = _compact_call()(filt.reshape(B * H * W))
    topidx = _sort_topk(vals.reshape(B, per_b), idxs.reshape(B, per_b))
    rows = score_map.reshape((B * H * W) // 8, 8)
    out8 = _sample_call()(rows, topidx.reshape(B * NSORT))
    return out8.reshape(B, NSORT, 8)[:, :NKP, :3]


# speedup vs baseline: 111.8654x; 1.0372x over previous
import functools

import jax
import jax.numpy as jnp
import numpy as np
from jax import lax
from jax.experimental import pallas as pl
from jax.experimental.pallas import tpu as pltpu
from jax.experimental.pallas import tpu_sc as plsc

B = 8
H = 512
W = 512
NKP = 1000
NSORT = 1024
NC = 2
NS = 16
NWORK = NC * NS
SLICE = (B * H * W) // NWORK
CAP = 4096
KP_PER_W = (B * NSORT) // NWORK
ROWS_PER_KP = 10
IDXLEN = KP_PER_W * ROWS_PER_KP
GCHUNK = 128
PAD_IDX = 1 << 30


def _nms_body(x_ref, o_ref):
    x = x_ref[0]
    ninf = jnp.full((H, 2), -jnp.inf, jnp.float32)
    xe = jnp.concatenate([ninf, x, ninf], axis=1)
    rm = x
    for i in (0, 1, 3, 4):
        rm = jnp.maximum(rm, xe[:, i:i + W])
    ninf_r = jnp.full((2, W), -jnp.inf, jnp.float32)
    re = jnp.concatenate([ninf_r, rm, ninf_r], axis=0)
    cm = rm
    for i in (0, 1, 3, 4):
        cm = jnp.maximum(cm, re[i:i + H, :])
    o_ref[0] = jnp.where(cm == x, x, 0.0)


_nms_call = pl.pallas_call(
    _nms_body,
    grid=(B,),
    in_specs=[pl.BlockSpec((1, H, W), lambda b: (b, 0, 0))],
    out_specs=pl.BlockSpec((1, H, W), lambda b: (b, 0, 0)),
    out_shape=jax.ShapeDtypeStruct((B, H, W), jnp.float32),
)


def _compact_body(filt_hbm, outv_hbm, outi_hbm, inbuf, vbuf, ibuf):
    wid = lax.axis_index("s") * NC + lax.axis_index("c")
    base = wid * SLICE
    pltpu.sync_copy(filt_hbm.at[pl.ds(base, SLICE)], inbuf)

    neg = jnp.full((16,), -jnp.inf, jnp.float32)
    padi = jnp.full((16,), PAD_IDX, jnp.int32)

    def fill(i, carry):
        vbuf[pl.ds(i * 16, 16)] = neg
        ibuf[pl.ds(i * 16, 16)] = padi
        return carry

    lax.fori_loop(0, (CAP + 16) // 16, fill, jnp.int32(0))

    iota16 = lax.iota(jnp.int32, 16)

    def body(i, pos):
        vs_ = [inbuf[pl.ds((i * 8 + g) * 16, 16)] for g in range(8)]
        ms_ = [v > 0.0 for v in vs_]
        ranks = [plsc.cumsum(m.astype(jnp.int32)) for m in ms_]
        cnts = [plsc.all_reduce_population_count(m) for m in ms_]
        for g in range(8):
            tgt = jnp.where(ms_[g], pos + ranks[g] - 1, CAP)
            idx = base + (i * 8 + g) * 16 + iota16
            plsc.store_scatter(vbuf, [tgt], vs_[g])
            plsc.store_scatter(ibuf, [tgt], idx)
            pos = jnp.minimum(pos + cnts[g], CAP - 16)
        return pos

    lax.fori_loop(0, SLICE // 128, body, jnp.full((16,), 0, jnp.int32))
    pltpu.sync_copy(vbuf.at[pl.ds(0, CAP)], outv_hbm.at[wid])
    pltpu.sync_copy(ibuf.at[pl.ds(0, CAP)], outi_hbm.at[wid])


@functools.cache
def _compact_call():
    return functools.partial(
        pl.kernel,
        out_type=[
            jax.ShapeDtypeStruct((NWORK, CAP), jnp.float32),
            jax.ShapeDtypeStruct((NWORK, CAP), jnp.int32),
        ],
        mesh=plsc.VectorSubcoreMesh(core_axis_name="c", subcore_axis_name="s"),
        compiler_params=pltpu.CompilerParams(needs_layout_passes=False),
        scratch_types=[
            pltpu.VMEM((SLICE,), jnp.float32),
            pltpu.VMEM((CAP + 16,), jnp.float32),
            pltpu.VMEM((CAP + 16,), jnp.int32),
        ],
    )(_compact_body)


NTOT = (NWORK // B) * CAP


def _roll(a, sh):
    sh %= a.shape[1]
    if sh == 0:
        return a
    return jnp.concatenate([a[:, -sh:], a[:, :-sh]], axis=1)


def _cmp_exchange(v, ix, j, desc):
    pos = lax.broadcasted_iota(jnp.int32, v.shape, 1)
    bit = (pos & j) != 0
    rvj = _roll(v, j)
    rij = _roll(ix, j)
    n = v.shape[1]
    pv = jnp.where(bit, rvj, _roll(rvj, n - 2 * j))
    pi = jnp.where(bit, rij, _roll(rij, n - 2 * j))
    beats = (v > pv) | ((v == pv) & (ix < pi))
    take_self = beats == (desc != bit)
    return jnp.where(take_self, v, pv), jnp.where(take_self, ix, pi)


def _make_phase1_call(ks):
    def body(v_ref, i_ref, ov_ref, oi_ref):
        v = v_ref[...]
        ix = i_ref[...]
        pos = lax.broadcasted_iota(jnp.int32, v.shape, 1)
        lpos = pos & (NSORT - 1)
        chunk_desc = (pos & (NTOT // 2)) == 0
        for k in ks:
            desc = ((lpos & k) == 0) == chunk_desc
            j = k // 2
            while j >= 1:
                v, ix = _cmp_exchange(v, ix, j, desc)
                j //= 2
        ov_ref[...] = v
        oi_ref[...] = ix

    return pl.pallas_call(
        body,
        out_shape=[jax.ShapeDtypeStruct((B, NTOT), jnp.float32),
                   jax.ShapeDtypeStruct((B, NTOT), jnp.int32)])


def _make_fold_merge_call(n):
    h = n // 2

    def body(v_ref, i_ref, ov_ref, oi_ref):
        a, ai = v_ref[:, :h], i_ref[:, :h]
        bv, bi = v_ref[:, h:], i_ref[:, h:]
        beats = (a > bv) | ((a == bv) & (ai < bi))
        v = jnp.where(beats, a, bv)
        ix = jnp.where(beats, ai, bi)
        pos = lax.broadcasted_iota(jnp.int32, (B, h), 1)
        desc = ((pos & (h // 2)) == 0) if h > NSORT else jnp.bool_(True)
        j = NSORT // 2
        while j >= 1:
            v, ix = _cmp_exchange(v, ix, j, desc)
            j //= 2
        ov_ref[...] = v
        oi_ref[...] = ix

    return pl.pallas_call(
        body,
        out_shape=[jax.ShapeDtypeStruct((B, h), jnp.float32),
                   jax.ShapeDtypeStruct((B, h), jnp.int32)])


_P1_GROUPS = [(2, 4, 8), (16, 32), (64,), (128,), (256,), (512,), (1024,)]


@functools.cache
def _sort_calls():
    p1 = [_make_phase1_call(ks) for ks in _P1_GROUPS]
    p2 = [_make_fold_merge_call(n) for n in (NTOT, NTOT // 2, NTOT // 4,
                                             NTOT // 8)]
    return p1, p2


def _sort_topk(vals, idxs):
    p1, p2 = _sort_calls()
    v, ix = vals, idxs
    for call in p1:
        v, ix = call(v, ix)
    for call in p2:
        v, ix = call(v, ix)
    return ix


def _sample_body(rows_hbm, kp_hbm, out_hbm, kpbuf, idxbuf, rowbuf, outbuf, sem):
    wid = lax.axis_index("s") * NC + lax.axis_index("c")
    b = wid // (NWORK // B)
    pltpu.sync_copy(kp_hbm.at[pl.ds(wid * KP_PER_W, KP_PER_W)], kpbuf)

    iota16 = lax.iota(jnp.int32, 16)

    def addr(t):
        kp = kpbuf[pl.ds(t * 16, 16)]
        y = (kp >> 9) & (H - 1)
        x = kp & (W - 1)
        s0 = jnp.clip(x - 2, 0, W - 5)
        r0w = jnp.minimum(s0 >> 3, W // 8 - 2)
        return kp, y, x, r0w

    def build_idx(t, carry):
        _, y, x, r0w = addr(t)
        for dy in range(5):
            ycl = jnp.clip(y + (dy - 2), 0, H - 1)
            r = (b * H + ycl) * (W // 8) + r0w
            for h in range(2):
                idxbuf[pl.ds(t * 160 + dy * 32 + h * 16, 16)] = r + h
        return carry

    lax.fori_loop(0, KP_PER_W // 16, build_idx, jnp.int32(0))

    cps = []
    for ch in range(IDXLEN // GCHUNK):
        cps.append(pltpu.async_copy(
            rows_hbm.at[idxbuf.at[pl.ds(ch * GCHUNK, GCHUNK)]],
            rowbuf.at[pl.ds(ch * GCHUNK, GCHUNK)],
            sem))
    for cp in cps:
        cp.wait()

    def sample(t, carry):
        _, y, x, r0w = addr(t)
        colbase = r0w * 8
        tb = t * 160
        xf = x.astype(jnp.float32)
        yf = y.astype(jnp.float32)

        def window(dyc, oc):
            ridx = tb + dyc * 32 + (oc >> 3) * 16 + iota16
            return plsc.load_gather(rowbuf, [ridx, oc & 7])

        xnum = jnp.zeros((16,), jnp.float32)
        ynum = jnp.zeros((16,), jnp.float32)
        den = jnp.zeros((16,), jnp.float32)
        for dy in range(-2, 3):
            yin = (y + dy >= 0) & (y + dy <= H - 1)
            for dx in range(-2, 3):
                xin = (x + dx >= 0) & (x + dx <= W - 1)
                oc = jnp.clip(x + dx, 0, W - 1) - colbase
                val = window(dy + 2, oc)
                e = jnp.where(yin & xin, jnp.exp(val * 5.0), 1.0)
                xnum = xnum + e * dx
                ynum = ynum + e * dy
                den = den + e
        kpx = xf + xnum / den
        kpy = yf + ynum / den

        def floor_ceil(u):
            ti = u.astype(jnp.int32)
            tf = ti.astype(jnp.float32)
            return ti - (u < tf).astype(jnp.int32), ti + (u > tf).astype(jnp.int32)

        fxi, cxi = floor_ceil(kpx)
        fyi, cyi = floor_ceil(kpy)
        fxc = jnp.clip(fxi, 0, W - 1)
        cxc = jnp.clip(cxi, 0, W - 1)
        fyc = jnp.clip(fyi, 0, H - 1)
        cyc = jnp.clip(cyi, 0, H - 1)

        def corner(cy, cx):
            return window(cy - y + 2, cx - colbase)

        c00 = corner(fyc, fxc)
        c10 = corner(fyc, cxc)
        c01 = corner(cyc, fxc)
        c11 = corner(cyc, cxc)
        fxf = fxc.astype(jnp.float32)
        cxf = cxc.astype(jnp.float32)
        fyf = fyc.astype(jnp.float32)
        cyf = cyc.astype(jnp.float32)
        conf = (c00 * (cxf - kpx) * (cyf - kpy)
                + c10 * (kpx - fxf) * (cyf - kpy)
                + c01 * (cxf - kpx) * (kpy - fyf)
                + c11 * (kpx - fxf) * (kpy - fyf))

        rows = t * 16 + iota16
        zero = jnp.zeros((16,), jnp.int32)
        plsc.store_scatter(outbuf, [rows, zero], kpx)
        plsc.store_scatter(outbuf, [rows, zero + 1], kpy)
        plsc.store_scatter(outbuf, [rows, zero + 2], conf)
        return carry

    lax.fori_loop(0, KP_PER_W // 16, sample, jnp.int32(0))
    pltpu.sync_copy(outbuf, out_hbm.at[pl.ds(wid * KP_PER_W, KP_PER_W)])


@functools.cache
def _sample_call():
    return functools.partial(
        pl.kernel,
        out_type=jax.ShapeDtypeStruct((B * NSORT, 8), jnp.float32),
        mesh=plsc.VectorSubcoreMesh(core_axis_name="c", subcore_axis_name="s"),
        compiler_params=pltpu.CompilerParams(
            needs_layout_passes=False, use_tc_tiling_on_sc=False),
        scratch_types=[
            pltpu.VMEM((KP_PER_W,), jnp.int32),
            pltpu.VMEM((IDXLEN,), jnp.int32),
            pltpu.VMEM((IDXLEN, 8), jnp.float32),
            pltpu.VMEM((KP_PER_W, 8), jnp.float32),
            pltpu.SemaphoreType.DMA,
        ],
    )(_sample_body)


def kernel(score_map):
    s = score_map.reshape(B, H, W)
    filt = _nms_call(s)
    per_b = (NWORK // B) * CAP
    vals, idxs = _compact_call()(filt.reshape(B * H * W))
    topidx = _sort_topk(vals.reshape(B, per_b), idxs.reshape(B, per_b))
    rows = score_map.reshape((B * H * W) // 8, 8)
    out8 = _sample_call()(rows, topidx.reshape(B * NSORT))
    return out8.reshape(B, NSORT, 8)[:, :NKP, :3]


# speedup vs baseline: 113.3738x; 1.0135x over previous
import functools

import jax
import jax.numpy as jnp
import numpy as np
from jax import lax
from jax.experimental import pallas as pl
from jax.experimental.pallas import tpu as pltpu
from jax.experimental.pallas import tpu_sc as plsc

B = 8
H = 512
W = 512
NKP = 1000
NSORT = 1024
NC = 2
NS = 16
NWORK = NC * NS
SLICE = (B * H * W) // NWORK
CAP = 4096
KP_PER_W = (B * NSORT) // NWORK
ROWS_PER_KP = 10
IDXLEN = KP_PER_W * ROWS_PER_KP
GCHUNK = 128
PAD_IDX = 1 << 30


NMS_BB = 4


def _nms_body(x_ref, o_ref):
    x = x_ref[...]
    ninf = jnp.full((NMS_BB, H, 2), -jnp.inf, jnp.float32)
    xe = jnp.concatenate([ninf, x, ninf], axis=2)
    rm = x
    for i in (0, 1, 3, 4):
        rm = jnp.maximum(rm, xe[:, :, i:i + W])
    ninf_r = jnp.full((NMS_BB, 2, W), -jnp.inf, jnp.float32)
    re = jnp.concatenate([ninf_r, rm, ninf_r], axis=1)
    cm = rm
    for i in (0, 1, 3, 4):
        cm = jnp.maximum(cm, re[:, i:i + H, :])
    o_ref[...] = jnp.where(cm == x, x, 0.0)


_nms_call = pl.pallas_call(
    _nms_body,
    grid=(B // NMS_BB,),
    in_specs=[pl.BlockSpec((NMS_BB, H, W), lambda b: (b, 0, 0))],
    out_specs=pl.BlockSpec((NMS_BB, H, W), lambda b: (b, 0, 0)),
    out_shape=jax.ShapeDtypeStruct((B, H, W), jnp.float32),
)


def _compact_body(filt_hbm, outv_hbm, outi_hbm, inbuf, vbuf, ibuf):
    wid = lax.axis_index("s") * NC + lax.axis_index("c")
    base = wid * SLICE
    pltpu.sync_copy(filt_hbm.at[pl.ds(base, SLICE)], inbuf)

    neg = jnp.full((16,), -jnp.inf, jnp.float32)
    padi = jnp.full((16,), PAD_IDX, jnp.int32)

    def fill(i, carry):
        vbuf[pl.ds(i * 16, 16)] = neg
        ibuf[pl.ds(i * 16, 16)] = padi
        return carry

    lax.fori_loop(0, (CAP + 16) // 16, fill, jnp.int32(0))

    iota16 = lax.iota(jnp.int32, 16)

    def body(i, pos):
        vs_ = [inbuf[pl.ds((i * 8 + g) * 16, 16)] for g in range(8)]
        ms_ = [v > 0.0 for v in vs_]
        ranks = [plsc.cumsum(m.astype(jnp.int32)) for m in ms_]
        cnts = [plsc.all_reduce_population_count(m) for m in ms_]
        for g in range(8):
            tgt = jnp.where(ms_[g], pos + ranks[g] - 1, CAP)
            idx = base + (i * 8 + g) * 16 + iota16
            plsc.store_scatter(vbuf, [tgt], vs_[g])
            plsc.store_scatter(ibuf, [tgt], idx)
            pos = jnp.minimum(pos + cnts[g], CAP - 16)
        return pos

    lax.fori_loop(0, SLICE // 128, body, jnp.full((16,), 0, jnp.int32))
    pltpu.sync_copy(vbuf.at[pl.ds(0, CAP)], outv_hbm.at[wid])
    pltpu.sync_copy(ibuf.at[pl.ds(0, CAP)], outi_hbm.at[wid])


@functools.cache
def _compact_call():
    return functools.partial(
        pl.kernel,
        out_type=[
            jax.ShapeDtypeStruct((NWORK, CAP), jnp.float32),
            jax.ShapeDtypeStruct((NWORK, CAP), jnp.int32),
        ],
        mesh=plsc.VectorSubcoreMesh(core_axis_name="c", subcore_axis_name="s"),
        compiler_params=pltpu.CompilerParams(needs_layout_passes=False),
        scratch_types=[
            pltpu.VMEM((SLICE,), jnp.float32),
            pltpu.VMEM((CAP + 16,), jnp.float32),
            pltpu.VMEM((CAP + 16,), jnp.int32),
        ],
    )(_compact_body)


NTOT = (NWORK // B) * CAP


def _roll(a, sh):
    sh %= a.shape[1]
    if sh == 0:
        return a
    return jnp.concatenate([a[:, -sh:], a[:, :-sh]], axis=1)


def _cmp_exchange(v, ix, j, desc):
    pos = lax.broadcasted_iota(jnp.int32, v.shape, 1)
    bit = (pos & j) != 0
    rvj = _roll(v, j)
    rij = _roll(ix, j)
    n = v.shape[1]
    pv = jnp.where(bit, rvj, _roll(rvj, n - 2 * j))
    pi = jnp.where(bit, rij, _roll(rij, n - 2 * j))
    beats = (v > pv) | ((v == pv) & (ix < pi))
    take_self = beats == (desc != bit)
    return jnp.where(take_self, v, pv), jnp.where(take_self, ix, pi)


def _make_phase1_call(ks):
    def body(v_ref, i_ref, ov_ref, oi_ref):
        v = v_ref[...]
        ix = i_ref[...]
        pos = lax.broadcasted_iota(jnp.int32, v.shape, 1)
        lpos = pos & (NSORT - 1)
        chunk_desc = (pos & (NTOT // 2)) == 0
        for k in ks:
            desc = ((lpos & k) == 0) == chunk_desc
            j = k // 2
            while j >= 1:
                v, ix = _cmp_exchange(v, ix, j, desc)
                j //= 2
        ov_ref[...] = v
        oi_ref[...] = ix

    return pl.pallas_call(
        body,
        out_shape=[jax.ShapeDtypeStruct((B, NTOT), jnp.float32),
                   jax.ShapeDtypeStruct((B, NTOT), jnp.int32)])


def _make_fold_merge_call(n):
    h = n // 2

    def body(v_ref, i_ref, ov_ref, oi_ref):
        a, ai = v_ref[:, :h], i_ref[:, :h]
        bv, bi = v_ref[:, h:], i_ref[:, h:]
        beats = (a > bv) | ((a == bv) & (ai < bi))
        v = jnp.where(beats, a, bv)
        ix = jnp.where(beats, ai, bi)
        pos = lax.broadcasted_iota(jnp.int32, (B, h), 1)
        desc = ((pos & (h // 2)) == 0) if h > NSORT else jnp.bool_(True)
        j = NSORT // 2
        while j >= 1:
            v, ix = _cmp_exchange(v, ix, j, desc)
            j //= 2
        ov_ref[...] = v
        oi_ref[...] = ix

    return pl.pallas_call(
        body,
        out_shape=[jax.ShapeDtypeStruct((B, h), jnp.float32),
                   jax.ShapeDtypeStruct((B, h), jnp.int32)])


_P1_GROUPS = [(2, 4, 8, 16), (32, 64), (128, 256), (512,), (1024,)]


@functools.cache
def _sort_calls():
    p1 = [_make_phase1_call(ks) for ks in _P1_GROUPS]
    p2 = [_make_fold_merge_call(n) for n in (NTOT, NTOT // 2, NTOT // 4,
                                             NTOT // 8)]
    return p1, p2


def _sort_topk(vals, idxs):
    p1, p2 = _sort_calls()
    v, ix = vals, idxs
    for call in p1:
        v, ix = call(v, ix)
    for call in p2:
        v, ix = call(v, ix)
    return ix


def _sample_body(rows_hbm, kp_hbm, out_hbm, kpbuf, idxbuf, rowbuf, outbuf, sem):
    wid = lax.axis_index("s") * NC + lax.axis_index("c")
    b = wid // (NWORK // B)
    pltpu.sync_copy(kp_hbm.at[pl.ds(wid * KP_PER_W, KP_PER_W)], kpbuf)

    iota16 = lax.iota(jnp.int32, 16)

    def addr(t):
        kp = kpbuf[pl.ds(t * 16, 16)]
        y = (kp >> 9) & (H - 1)
        x = kp & (W - 1)
        s0 = jnp.clip(x - 2, 0, W - 5)
        r0w = jnp.minimum(s0 >> 3, W // 8 - 2)
        return kp, y, x, r0w

    def build_idx(t, carry):
        _, y, x, r0w = addr(t)
        for dy in range(5):
            ycl = jnp.clip(y + (dy - 2), 0, H - 1)
            r = (b * H + ycl) * (W // 8) + r0w
            for h in range(2):
                idxbuf[pl.ds(t * 160 + dy * 32 + h * 16, 16)] = r + h
        return carry

    lax.fori_loop(0, KP_PER_W // 16, build_idx, jnp.int32(0))

    cps = []
    for ch in range(IDXLEN // GCHUNK):
        cps.append(pltpu.async_copy(
            rows_hbm.at[idxbuf.at[pl.ds(ch * GCHUNK, GCHUNK)]],
            rowbuf.at[pl.ds(ch * GCHUNK, GCHUNK)],
            sem))
    for cp in cps:
        cp.wait()

    def sample(t, carry):
        _, y, x, r0w = addr(t)
        colbase = r0w * 8
        tb = t * 160
        xf = x.astype(jnp.float32)
        yf = y.astype(jnp.float32)

        def window(dyc, oc):
            ridx = tb + dyc * 32 + (oc >> 3) * 16 + iota16
            return plsc.load_gather(rowbuf, [ridx, oc & 7])

        xnum = jnp.zeros((16,), jnp.float32)
        ynum = jnp.zeros((16,), jnp.float32)
        den = jnp.zeros((16,), jnp.float32)
        for dy in range(-2, 3):
            yin = (y + dy >= 0) & (y + dy <= H - 1)
            for dx in range(-2, 3):
                xin = (x + dx >= 0) & (x + dx <= W - 1)
                oc = jnp.clip(x + dx, 0, W - 1) - colbase
                val = window(dy + 2, oc)
                e = jnp.where(yin & xin, jnp.exp(val * 5.0), 1.0)
                xnum = xnum + e * dx
                ynum = ynum + e * dy
                den = den + e
        kpx = xf + xnum / den
        kpy = yf + ynum / den

        def floor_ceil(u):
            ti = u.astype(jnp.int32)
            tf = ti.astype(jnp.float32)
            return ti - (u < tf).astype(jnp.int32), ti + (u > tf).astype(jnp.int32)

        fxi, cxi = floor_ceil(kpx)
        fyi, cyi = floor_ceil(kpy)
        fxc = jnp.clip(fxi, 0, W - 1)
        cxc = jnp.clip(cxi, 0, W - 1)
        fyc = jnp.clip(fyi, 0, H - 1)
        cyc = jnp.clip(cyi, 0, H - 1)

        def corner(cy, cx):
            return window(cy - y + 2, cx - colbase)

        c00 = corner(fyc, fxc)
        c10 = corner(fyc, cxc)
        c01 = corner(cyc, fxc)
        c11 = corner(cyc, cxc)
        fxf = fxc.astype(jnp.float32)
        cxf = cxc.astype(jnp.float32)
        fyf = fyc.astype(jnp.float32)
        cyf = cyc.astype(jnp.float32)
        conf = (c00 * (cxf - kpx) * (cyf - kpy)
                + c10 * (kpx - fxf) * (cyf - kpy)
                + c01 * (cxf - kpx) * (kpy - fyf)
                + c11 * (kpx - fxf) * (kpy - fyf))

        rows = t * 16 + iota16
        zero = jnp.zeros((16,), jnp.int32)
        plsc.store_scatter(outbuf, [rows, zero], kpx)
        plsc.store_scatter(outbuf, [rows, zero + 1], kpy)
        plsc.store_scatter(outbuf, [rows, zero + 2], conf)
        return carry

    lax.fori_loop(0, KP_PER_W // 16, sample, jnp.int32(0))
    pltpu.sync_copy(outbuf, out_hbm.at[pl.ds(wid * KP_PER_W, KP_PER_W)])


@functools.cache
def _sample_call():
    return functools.partial(
        pl.kernel,
        out_type=jax.ShapeDtypeStruct((B * NSORT, 8), jnp.float32),
        mesh=plsc.VectorSubcoreMesh(core_axis_name="c", subcore_axis_name="s"),
        compiler_params=pltpu.CompilerParams(
            needs_layout_passes=False, use_tc_tiling_on_sc=False),
        scratch_types=[
            pltpu.VMEM((KP_PER_W,), jnp.int32),
            pltpu.VMEM((IDXLEN,), jnp.int32),
            pltpu.VMEM((IDXLEN, 8), jnp.float32),
            pltpu.VMEM((KP_PER_W, 8), jnp.float32),
            pltpu.SemaphoreType.DMA,
        ],
    )(_sample_body)


def kernel(score_map):
    s = score_map.reshape(B, H, W)
    filt = _nms_call(s)
    per_b = (NWORK // B) * CAP
    vals, idxs = _compact_call()(filt.reshape(B * H * W))
    topidx = _sort_topk(vals.reshape(B, per_b), idxs.reshape(B, per_b))
    rows = score_map.reshape((B * H * W) // 8, 8)
    out8 = _sample_call()(rows, topidx.reshape(B * NSORT))
    return out8.reshape(B, NSORT, 8)[:, :NKP, :3]
